# baseline (speedup 1.0000x reference)
import jax
import jax.numpy as jnp
from jax import lax
from jax.experimental import pallas as pl
from jax.experimental.pallas import tpu as pltpu
from jax.experimental.pallas import tpu_sc as plsc

_D = 64
_BATCH = 16384
_MARGIN = 1.0

_NC = 2
_NS = 16
_NW = _NC * _NS
_B_PER_W = _BATCH // _NW
_CHUNK = 128
_NCHUNK = _B_PER_W // _CHUNK
_GROUPS = _CHUNK // 16


def _sc_kernel(table, pos, neg, out, trip_p, trip_n, ph_i, pt_i, nh_i, nt_i,
               bufs0, bufs1, loss_v, sem0, sem1):
    wid = lax.axis_index("s") * _NC + lax.axis_index("c")
    base = wid * _B_PER_W
    pltpu.sync_copy(pos.at[pl.ds(base, _B_PER_W)], trip_p)
    pltpu.sync_copy(neg.at[pl.ds(base, _B_PER_W)], trip_n)

    lanes = lax.iota(jnp.int32, 16)
    c0 = jnp.zeros((16,), jnp.int32)
    c2 = jnp.full((16,), 2, jnp.int32)
    for g in range(_B_PER_W // 16):
        rows = g * 16 + lanes
        ph_i[pl.ds(g * 16, 16)] = plsc.load_gather(trip_p, [rows, c0])
        pt_i[pl.ds(g * 16, 16)] = plsc.load_gather(trip_p, [rows, c2])
        nh_i[pl.ds(g * 16, 16)] = plsc.load_gather(trip_n, [rows, c0])
        nt_i[pl.ds(g * 16, 16)] = plsc.load_gather(trip_n, [rows, c2])

    idxs = (ph_i, pt_i, nh_i, nt_i)
    slot_bufs = (bufs0, bufs1)
    slot_sems = (sem0, sem1)

    def fire(c, slot):
        return [
            pltpu.async_copy(table.at[idxs[k].at[pl.ds(c * _CHUNK, _CHUNK)]],
                             slot_bufs[slot][k], slot_sems[slot])
            for k in range(4)
        ]

    inflight = {0: fire(0, 0), 1: fire(1, 1)}
    loss = jnp.zeros((16,), jnp.float32)

    for c in range(_NCHUNK):
        slot = c % 2
        for cp in inflight.pop(c):
            cp.wait()
        ph_r, pt_r, nh_r, nt_r = slot_bufs[slot]

        def group_body(g, acc):
            rows = g * 16 + lanes
            pos_s = jnp.zeros((16,), jnp.float32)
            neg_s = jnp.zeros((16,), jnp.float32)
            for d in range(_D):
                col = jnp.full((16,), d, jnp.int32)
                dp = (plsc.load_gather(ph_r, [rows, col])
                      - plsc.load_gather(pt_r, [rows, col]))
                dn = (plsc.load_gather(nh_r, [rows, col])
                      - plsc.load_gather(nt_r, [rows, col]))
                pos_s = pos_s + dp * dp
                neg_s = neg_s + dn * dn
            return acc + jnp.maximum(pos_s - neg_s + _MARGIN, 0.0)

        loss = lax.fori_loop(0, _GROUPS, group_body, loss)
        if c + 2 < _NCHUNK:
            inflight[c + 2] = fire(c + 2, slot)

    loss_v[...] = loss
    pltpu.sync_copy(loss_v, out.at[wid])


def _reduce_kernel(x_ref, o_ref):
    o_ref[...] = jnp.sum(x_ref[...], axis=(0, 1), keepdims=True) * (1.0 / _BATCH)


@jax.jit
def kernel(batch_positives, batch_negatives, entity_embeddings):
    mesh = plsc.VectorSubcoreMesh(core_axis_name="c", subcore_axis_name="s")
    row_buf = pltpu.VMEM((_CHUNK, _D), jnp.float32)
    partials = pl.kernel(
        _sc_kernel,
        out_type=jax.ShapeDtypeStruct((_NW, 16), jnp.float32),
        mesh=mesh,
        compiler_params=pltpu.CompilerParams(
            needs_layout_passes=False, use_tc_tiling_on_sc=False),
        scratch_types=[
            pltpu.VMEM((_B_PER_W, 3), jnp.int32),
            pltpu.VMEM((_B_PER_W, 3), jnp.int32),
            pltpu.VMEM((_B_PER_W,), jnp.int32),
            pltpu.VMEM((_B_PER_W,), jnp.int32),
            pltpu.VMEM((_B_PER_W,), jnp.int32),
            pltpu.VMEM((_B_PER_W,), jnp.int32),
            (row_buf, row_buf, row_buf, row_buf),
            (row_buf, row_buf, row_buf, row_buf),
            pltpu.VMEM((16,), jnp.float32),
            pltpu.SemaphoreType.DMA,
            pltpu.SemaphoreType.DMA,
        ],
    )(entity_embeddings, batch_positives, batch_negatives)

    loss = pl.pallas_call(
        _reduce_kernel,
        out_shape=jax.ShapeDtypeStruct((1, 1), jnp.float32),
    )(partials)
    return loss[0, 0]


# speedup vs baseline: 1.7728x; 1.7728x over previous
import jax
import jax.numpy as jnp
from jax import lax
from jax.experimental import pallas as pl
from jax.experimental.pallas import tpu as pltpu
from jax.experimental.pallas import tpu_sc as plsc

_D = 64
_BATCH = 16384
_MARGIN = 1.0

_NC = 2
_NS = 16
_NW = _NC * _NS
_B_PER_W = _BATCH // _NW
_CHUNK = 64
_NCHUNK = _B_PER_W // _CHUNK
_GROUPS = _CHUNK // 16


def _sc_kernel(table, pos, neg, out, trip_p, trip_n,
               bufs0, bufs1, accv, loss_v, sem0, sem1):
    wid = lax.axis_index("s") * _NC + lax.axis_index("c")
    base = wid * _B_PER_W
    pltpu.sync_copy(pos.at[pl.ds(base * 3, _B_PER_W * 3)],
                    trip_p.at[pl.ds(0, _B_PER_W * 3)])
    pltpu.sync_copy(neg.at[pl.ds(base * 3, _B_PER_W * 3)],
                    trip_n.at[pl.ds(0, _B_PER_W * 3)])

    lanes = lax.iota(jnp.int32, 16)
    slot_bufs = (bufs0, bufs1)
    slot_sems = (sem0, sem1)

    def fire(c, slot):
        bufs = slot_bufs[slot]
        sem = slot_sems[slot]

        def issue(j, carry):
            j3 = c * _CHUNK * 3 + j * 3
            vp = trip_p[pl.ds(j3, 16)]
            vn = trip_n[pl.ds(j3, 16)]
            for k, e in enumerate((vp[0], vp[2], vn[0], vn[2])):
                pltpu.async_copy(table.at[pl.ds(e, 1)],
                                 bufs[k].at[pl.ds(j, 1)], sem)
            return carry

        lax.fori_loop(0, _CHUNK, issue, 0)

    def drain(slot):
        for k in range(4):
            pltpu.make_async_copy(table.at[pl.ds(0, _CHUNK)],
                                  slot_bufs[slot][k], slot_sems[slot]).wait()

    fire(0, 0)
    fire(1, 1)
    loss = jnp.zeros((16,), jnp.float32)

    perm_lo = (2 * lanes) % 16
    perm_hi = perm_lo + 1
    mask_lo = lanes < 8

    def combine(x, y):
        xa = (jnp.take_along_axis(x, perm_lo, axis=0)
              + jnp.take_along_axis(x, perm_hi, axis=0))
        ya = (jnp.take_along_axis(y, perm_lo, axis=0)
              + jnp.take_along_axis(y, perm_hi, axis=0))
        return jnp.where(mask_lo, xa, ya)

    for c in range(_NCHUNK):
        slot = c % 2
        drain(slot)
        bufs = slot_bufs[slot]

        def group_body(g, acc):
            gbase = g * 16
            for r in range(16):
                j = gbase + r
                av = None
                for k in range(4):
                    dp = (bufs[0][j, pl.ds(k * 16, 16)]
                          - bufs[1][j, pl.ds(k * 16, 16)])
                    dn = (bufs[2][j, pl.ds(k * 16, 16)]
                          - bufs[3][j, pl.ds(k * 16, 16)])
                    term = dp * dp - dn * dn
                    av = term if av is None else av + term
                accv[r, pl.ds(0, 16)] = av
            accs = [accv[r, pl.ds(0, 16)] for r in range(16)]
            while len(accs) > 1:
                accs = [combine(accs[i], accs[i + 1])
                        for i in range(0, len(accs), 2)]
            return acc + jnp.maximum(accs[0] + _MARGIN, 0.0)

        loss = lax.fori_loop(0, _GROUPS, group_body, loss)
        if c + 2 < _NCHUNK:
            fire(c + 2, slot)

    for i in range(8):
        loss_v[pl.ds(i * 16, 16)] = loss if i == 0 else jnp.zeros(
            (16,), jnp.float32)
    pltpu.sync_copy(loss_v, out.at[wid])


def _reduce_kernel(x_ref, o_ref):
    o_ref[...] = jnp.sum(x_ref[...], axis=(0, 1), keepdims=True) * (1.0 / _BATCH)


@jax.jit
def kernel(batch_positives, batch_negatives, entity_embeddings):
    pos = batch_positives.reshape(-1)
    neg = batch_negatives.reshape(-1)

    mesh = plsc.VectorSubcoreMesh(core_axis_name="c", subcore_axis_name="s")
    row_buf = pltpu.VMEM((_CHUNK, _D), jnp.float32)
    partials = pl.kernel(
        _sc_kernel,
        out_type=jax.ShapeDtypeStruct((_NW, 128), jnp.float32),
        mesh=mesh,
        compiler_params=pltpu.CompilerParams(
            needs_layout_passes=False, use_tc_tiling_on_sc=True),
        scratch_types=[
            pltpu.VMEM((_B_PER_W * 3 + 64,), jnp.int32),
            pltpu.VMEM((_B_PER_W * 3 + 64,), jnp.int32),
            (row_buf, row_buf, row_buf, row_buf),
            (row_buf, row_buf, row_buf, row_buf),
            pltpu.VMEM((16, 16), jnp.float32),
            pltpu.VMEM((128,), jnp.float32),
            pltpu.SemaphoreType.DMA,
            pltpu.SemaphoreType.DMA,
        ],
    )(entity_embeddings, pos, neg)

    loss = pl.pallas_call(
        _reduce_kernel,
        out_shape=jax.ShapeDtypeStruct((1, 1), jnp.float32),
    )(partials)
    return loss[0, 0]


# speedup vs baseline: 2.6561x; 1.4983x over previous
import jax
import jax.numpy as jnp
from jax import lax
from jax.experimental import pallas as pl
from jax.experimental.pallas import tpu as pltpu
from jax.experimental.pallas import tpu_sc as plsc

_D = 64
_BATCH = 16384
_MARGIN = 1.0

_NC = 2
_NS = 16
_NW = _NC * _NS
_B_PER_W = _BATCH // _NW
_CHUNK = 64
_NCHUNK = _B_PER_W // _CHUNK
_GROUPS = _CHUNK // 16


def _sc_kernel(table, pos, neg, out, trip_p, trip_n,
               bufs0, bufs1, accv, loss_v, sem0, sem1):
    wid = lax.axis_index("s") * _NC + lax.axis_index("c")
    base = wid * _B_PER_W
    pltpu.sync_copy(pos.at[pl.ds(base * 3, _B_PER_W * 3)],
                    trip_p.at[pl.ds(0, _B_PER_W * 3)])
    pltpu.sync_copy(neg.at[pl.ds(base * 3, _B_PER_W * 3)],
                    trip_n.at[pl.ds(0, _B_PER_W * 3)])

    lanes = lax.iota(jnp.int32, 16)
    slot_bufs = (bufs0, bufs1)
    slot_sems = (sem0, sem1)

    def fire(c, slot):
        bufs = slot_bufs[slot]
        sem = slot_sems[slot]

        def issue(j, carry):
            j3 = c * _CHUNK * 3 + j * 3
            vp = trip_p[pl.ds(j3, 16)]
            vn = trip_n[pl.ds(j3, 16)]
            for k, e in enumerate((vp[0], vp[2], vn[0], vn[2])):
                pltpu.async_copy(table.at[pl.ds(e, 1)],
                                 bufs[k].at[pl.ds(j, 1)], sem)
            return carry

        lax.fori_loop(0, _CHUNK, issue, 0)

    def drain(slot):
        for k in range(4):
            pltpu.make_async_copy(table.at[pl.ds(0, _CHUNK)],
                                  slot_bufs[slot][k], slot_sems[slot]).wait()

    fire(0, 0)
    fire(1, 1)
    loss = jnp.zeros((16,), jnp.float32)

    perm_lo = (2 * lanes) % 16
    perm_hi = perm_lo + 1
    mask_lo = lanes < 8

    def combine(x, y):
        xa = (jnp.take_along_axis(x, perm_lo, axis=0)
              + jnp.take_along_axis(x, perm_hi, axis=0))
        ya = (jnp.take_along_axis(y, perm_lo, axis=0)
              + jnp.take_along_axis(y, perm_hi, axis=0))
        return jnp.where(mask_lo, xa, ya)

    for c in range(_NCHUNK):
        slot = c % 2
        drain(slot)
        bufs = slot_bufs[slot]

        def group_body(g, acc):
            gbase = g * 16
            for r in range(16):
                j = gbase + r
                av = None
                for k in range(4):
                    dp = (bufs[0][j, pl.ds(k * 16, 16)]
                          - bufs[1][j, pl.ds(k * 16, 16)])
                    dn = (bufs[2][j, pl.ds(k * 16, 16)]
                          - bufs[3][j, pl.ds(k * 16, 16)])
                    term = dp * dp - dn * dn
                    av = term if av is None else av + term
                accv[r, pl.ds(0, 16)] = av
            accs = [accv[r, pl.ds(0, 16)] for r in range(16)]
            while len(accs) > 1:
                accs = [combine(accs[i], accs[i + 1])
                        for i in range(0, len(accs), 2)]
            return acc + jnp.maximum(accs[0] + _MARGIN, 0.0)

        loss = lax.fori_loop(0, _GROUPS, group_body, loss)
        if c + 2 < _NCHUNK:
            fire(c + 2, slot)

    for i in range(8):
        loss_v[pl.ds(i * 16, 16)] = loss if i == 0 else jnp.zeros(
            (16,), jnp.float32)
    pltpu.sync_copy(loss_v, out.at[wid])


def _reduce_kernel(x_ref, o_ref):
    o_ref[...] = jnp.sum(x_ref[...], axis=(0, 1), keepdims=True) * (1.0 / _BATCH)


@jax.jit
def kernel(batch_positives, batch_negatives, entity_embeddings):
    pos = batch_positives.reshape(-1)
    neg = batch_negatives.reshape(-1)
    table = lax.transpose(lax.optimization_barrier(entity_embeddings.T), (1, 0))

    mesh = plsc.VectorSubcoreMesh(core_axis_name="c", subcore_axis_name="s")
    row_buf = pltpu.VMEM((_CHUNK, _D), jnp.float32)
    partials = pl.kernel(
        _sc_kernel,
        out_type=jax.ShapeDtypeStruct((_NW, 128), jnp.float32),
        mesh=mesh,
        compiler_params=pltpu.CompilerParams(
            needs_layout_passes=False, use_tc_tiling_on_sc=True),
        scratch_types=[
            pltpu.VMEM((_B_PER_W * 3 + 64,), jnp.int32),
            pltpu.VMEM((_B_PER_W * 3 + 64,), jnp.int32),
            (row_buf, row_buf, row_buf, row_buf),
            (row_buf, row_buf, row_buf, row_buf),
            pltpu.VMEM((16, 16), jnp.float32),
            pltpu.VMEM((128,), jnp.float32),
            pltpu.SemaphoreType.DMA,
            pltpu.SemaphoreType.DMA,
        ],
    )(table, pos, neg)

    loss = pl.pallas_call(
        _reduce_kernel,
        out_shape=jax.ShapeDtypeStruct((1, 1), jnp.float32),
    )(partials)
    return loss[0, 0]
